# baseline (speedup 1.0000x reference)
import jax
import jax.numpy as jnp
from jax.experimental import pallas as pl
from jax.experimental.pallas import tpu as pltpu

NUM_NODES = 2048
IN_DIM = 128
HIDDEN_DIM = 128
OUT_DIM = 128
HEADS = 4
BATCH = 2
F = HIDDEN_DIM // HEADS
FE = F + 1
JB = 256
NJ = NUM_NODES // JB


def _wext_t(W):
    K = W.shape[0]
    Wr = W.reshape(K, HEADS, F).transpose(1, 2, 0)
    return jnp.pad(Wr, ((0, 0), (0, 1), (0, 0))).astype(jnp.float32)


def _dot_t(a, b):
    return jax.lax.dot_general(a, b, (((1,), (1,)), ((), ())),
                               preferred_element_type=jnp.float32)


def _normalize(emb):
    return emb / (jnp.sqrt(jnp.sum(emb * emb, axis=1, keepdims=True)) + 1e-12)


def _fused_body(emb_ref, x_ref, pW_ref, pb_ref,
                W1eT_ref, A1s_ref, A1d_ref, W1as_ref,
                W2eT_ref, A2s_ref, A2d_ref,
                b1_ref, b2_ref, out_ref,
                mask_scr, u1_scr, u2_scr, sb_scr, nd_scr, d1_scr, d2_scr,
                h1_scr, nemb_scr):
    pid = pl.program_id(0)
    r = pid % NJ
    b = r // HEADS
    h = r % HEADS
    ones_row = (jnp.arange(FE) == F).astype(jnp.float32)[:, None]

    def store_factors(whxT, s_row, d_row, s_col):
        smax = jnp.max(s_row)
        dmax = jnp.max(d_row)
        ib, ih = pl.ds(b, 1), pl.ds(h, 1)
        u1_scr[ib, ih] = (jnp.exp(s_row - smax)
                          * whxT).astype(jnp.bfloat16)[None, None]
        u2_scr[ib, ih] = (jnp.exp(0.2 * s_row - smax)
                          * whxT).astype(jnp.bfloat16)[None, None]
        sb_scr[ib, ih] = jnp.broadcast_to(
            s_col, (NUM_NODES, JB))[None, None]
        nd_scr[ib, ih] = jnp.reshape(-d_row, (NJ, JB))[None, None]
        d1_scr[ib, ih] = jnp.reshape(jnp.exp(d_row - dmax),
                                     (NJ, JB))[None, None]
        d2_scr[ib, ih] = jnp.reshape(jnp.exp(0.2 * d_row - dmax),
                                     (NJ, JB))[None, None]

    def attn(j, first_layer):
        maskT = mask_scr[pl.ds(j, 1)][0]
        zero = jnp.zeros((), jnp.bfloat16)
        jb = pl.ds(j, 1)
        for bb in range(BATCH):
            outs = []
            for hh in range(HEADS):
                cond = sb_scr[bb, hh] >= nd_scr[bb, hh, jb]
                P = jnp.where(cond, maskT, zero)
                Q = maskT - P
                A = jnp.dot(u1_scr[bb, hh], P,
                            preferred_element_type=jnp.float32)
                Bm = jnp.dot(u2_scr[bb, hh], Q,
                             preferred_element_type=jnp.float32)
                R = (d1_scr[bb, hh, jb] * A
                     + d2_scr[bb, hh, jb] * Bm)
                outs.append(R[:F, :] / R[F:FE, :])
            o = jnp.concatenate(outs, axis=0)
            if first_layer:
                o = jnp.maximum(o + b1_ref[...], 0.0)
                h1_scr[pl.ds(j, 1), pl.ds(bb, 1)] = o[None, None]
            else:
                out_ref[bb] = o + b2_ref[...]

    @pl.when(pid < NJ)
    def _phase_mask_prep1():
        @pl.when(pid == 0)
        def _():
            nemb_scr[...] = _normalize(emb_ref[...])
        nf = nemb_scr[...]
        nb = nemb_scr[pl.ds(r * JB, JB), :]
        mask_scr[pl.ds(r, 1)] = (_dot_t(nf, nb) > 0.5).astype(
            jnp.bfloat16)[None]
        hb = jnp.dot(x_ref[pl.ds(b, 1)][0], pW_ref[...],
                     preferred_element_type=jnp.float32) + pb_ref[...][None, :]
        whxT = _dot_t(W1eT_ref[pl.ds(h, 1)][0], hb) + ones_row
        s_row = jnp.dot(A1s_ref[pl.ds(h, 1)][0], whxT,
                        preferred_element_type=jnp.float32)
        d_row = jnp.dot(A1d_ref[pl.ds(h, 1)][0], whxT,
                        preferred_element_type=jnp.float32)
        s_col = jnp.dot(hb, W1as_ref[pl.ds(h, 1)][0],
                        preferred_element_type=jnp.float32)
        store_factors(whxT, s_row, d_row, s_col)

    @pl.when(jnp.logical_and(pid >= NJ, pid < 2 * NJ))
    def _phase_attn1():
        attn(r, True)

    @pl.when(jnp.logical_and(pid >= 2 * NJ, pid < 3 * NJ))
    def _phase_prep2():
        parts = [h1_scr[pl.ds(jj, 1), pl.ds(b, 1)][0, 0] for jj in range(NJ)]
        hbT = jnp.concatenate(parts, axis=1)
        whxT = jnp.dot(W2eT_ref[pl.ds(h, 1)][0], hbT,
                       preferred_element_type=jnp.float32) + ones_row
        s_row = jnp.dot(A2s_ref[pl.ds(h, 1)][0], whxT,
                        preferred_element_type=jnp.float32)
        d_row = jnp.dot(A2d_ref[pl.ds(h, 1)][0], whxT,
                        preferred_element_type=jnp.float32)
        s_col = jnp.reshape(s_row, (NUM_NODES, 1))
        store_factors(whxT, s_row, d_row, s_col)

    @pl.when(pid >= 3 * NJ)
    def _phase_attn2():
        attn(r, False)


def kernel(x, embedding, proj_W, proj_b, W1, a1_src, a1_dst, b1,
           W2, a2_src, a2_dst, b2):
    N = NUM_NODES
    W1eT, W2eT = _wext_t(W1), _wext_t(W2)
    A1s = jnp.pad(a1_src, ((0, 0), (0, 1)))[:, None, :]
    A2s = jnp.pad(a2_src, ((0, 0), (0, 1)))[:, None, :]
    A1d = jnp.pad(a1_dst, ((0, 0), (0, 1)))[:, None, :]
    A2d = jnp.pad(a2_dst, ((0, 0), (0, 1)))[:, None, :]
    W1as = jnp.einsum('khf,hf->hk', W1.reshape(IN_DIM, HEADS, F), a1_src)
    W1as1 = W1as[:, :, None]

    full = lambda *shape: pl.BlockSpec(shape, lambda p: (0,) * len(shape))
    out_t = pl.pallas_call(
        _fused_body,
        grid=(4 * NJ,),
        in_specs=[
            full(N, IN_DIM),
            full(BATCH, N, IN_DIM),
            full(IN_DIM, IN_DIM),
            full(IN_DIM),
            full(HEADS, FE, IN_DIM),
            full(HEADS, 1, FE),
            full(HEADS, 1, FE),
            full(HEADS, IN_DIM, 1),
            full(HEADS, FE, HIDDEN_DIM),
            full(HEADS, 1, FE),
            full(HEADS, 1, FE),
            full(HIDDEN_DIM, 1),
            full(HIDDEN_DIM, 1),
        ],
        out_specs=pl.BlockSpec(
            (BATCH, HIDDEN_DIM, JB),
            lambda p: (0, 0, jnp.maximum(p - 3 * NJ, 0))),
        out_shape=jax.ShapeDtypeStruct((BATCH, HIDDEN_DIM, N), jnp.float32),
        scratch_shapes=[
            pltpu.VMEM((NJ, N, JB), jnp.bfloat16),
            pltpu.VMEM((BATCH, HEADS, FE, N), jnp.bfloat16),
            pltpu.VMEM((BATCH, HEADS, FE, N), jnp.bfloat16),
            pltpu.VMEM((BATCH, HEADS, N, JB), jnp.float32),
            pltpu.VMEM((BATCH, HEADS, NJ, JB), jnp.float32),
            pltpu.VMEM((BATCH, HEADS, NJ, JB), jnp.float32),
            pltpu.VMEM((BATCH, HEADS, NJ, JB), jnp.float32),
            pltpu.VMEM((NJ, BATCH, HIDDEN_DIM, JB), jnp.float32),
            pltpu.VMEM((N, IN_DIM), jnp.float32),
        ],
    )(embedding, x, proj_W, proj_b,
      W1eT, A1s, A1d, W1as1,
      W2eT, A2s, A2d,
      b1[:, None], b2[:, None])
    return jnp.swapaxes(out_t, 1, 2)


# speedup vs baseline: 1.0612x; 1.0612x over previous
import jax
import jax.numpy as jnp
from jax.experimental import pallas as pl
from jax.experimental.pallas import tpu as pltpu

NUM_NODES = 2048
IN_DIM = 128
HIDDEN_DIM = 128
OUT_DIM = 128
HEADS = 4
BATCH = 2
F = HIDDEN_DIM // HEADS
FE = F + 1
JB = 256
NJ = NUM_NODES // JB


def _wext_t(W):
    K = W.shape[0]
    Wr = W.reshape(K, HEADS, F).transpose(1, 2, 0)
    return jnp.pad(Wr, ((0, 0), (0, 1), (0, 0))).astype(jnp.float32)


def _dot_t(a, b):
    return jax.lax.dot_general(a, b, (((1,), (1,)), ((), ())),
                               preferred_element_type=jnp.float32)


def _normalize(emb):
    return emb / (jnp.sqrt(jnp.sum(emb * emb, axis=1, keepdims=True)) + 1e-12)


def _fused_body(emb_ref, x_ref, pW_ref, pb_ref,
                W1eT_ref, A1s_ref, A1d_ref, W1as_ref,
                W2eT_ref, A2s_ref, A2d_ref,
                b1_ref, b2_ref, out_ref,
                mask_scr, u1_scr, u2_scr, sc_scr, nd_scr, d1_scr, d2_scr,
                h1_scr, nemb_scr):
    pid = pl.program_id(0)
    r = pid % NJ
    b = r // HEADS
    h = r % HEADS
    ones_row = (jnp.arange(FE) == F).astype(jnp.float32)[:, None]

    def store_factors(whxT, s_row, d_row, s_col):
        smax = jnp.max(s_row)
        dmax = jnp.max(d_row)
        ib, ih = pl.ds(b, 1), pl.ds(h, 1)
        u1_scr[ib, ih] = (jnp.exp(s_row - smax)
                          * whxT).astype(jnp.bfloat16)[None, None]
        u2_scr[ib, ih] = (jnp.exp(0.2 * s_row - smax)
                          * whxT).astype(jnp.bfloat16)[None, None]
        sc_scr[ib, ih] = s_col[None, None]
        nd_scr[ib, ih] = jnp.reshape(-d_row, (NJ, JB))[None, None]
        d1_scr[ib, ih] = jnp.reshape(jnp.exp(d_row - dmax),
                                     (NJ, JB))[None, None]
        d2_scr[ib, ih] = jnp.reshape(jnp.exp(0.2 * d_row - dmax),
                                     (NJ, JB))[None, None]

    def attn(j, first_layer):
        maskT = mask_scr[pl.ds(j, 1)][0]
        zero = jnp.zeros((), jnp.bfloat16)
        jb = pl.ds(j, 1)
        for bb in range(BATCH):
            outs = []
            for hh in range(HEADS):
                cond = sc_scr[bb, hh] >= nd_scr[bb, hh, jb]
                P = jnp.where(cond, maskT, zero)
                Q = maskT - P
                A = jnp.dot(u1_scr[bb, hh], P,
                            preferred_element_type=jnp.float32)
                Bm = jnp.dot(u2_scr[bb, hh], Q,
                             preferred_element_type=jnp.float32)
                R = (d1_scr[bb, hh, jb] * A
                     + d2_scr[bb, hh, jb] * Bm)
                outs.append(R[:F, :] / R[F:FE, :])
            o = jnp.concatenate(outs, axis=0)
            if first_layer:
                o = jnp.maximum(o + b1_ref[...], 0.0)
                h1_scr[pl.ds(j, 1), pl.ds(bb, 1)] = o[None, None]
            else:
                out_ref[bb] = o + b2_ref[...]

    @pl.when(pid < NJ)
    def _phase_mask_prep1():
        @pl.when(pid == 0)
        def _():
            nemb_scr[...] = _normalize(emb_ref[...])
        nf = nemb_scr[...]
        nb = nemb_scr[pl.ds(r * JB, JB), :]
        mask_scr[pl.ds(r, 1)] = (_dot_t(nf, nb) > 0.5).astype(
            jnp.bfloat16)[None]
        hb = jnp.dot(x_ref[pl.ds(b, 1)][0], pW_ref[...],
                     preferred_element_type=jnp.float32) + pb_ref[...][None, :]
        whxT = _dot_t(W1eT_ref[pl.ds(h, 1)][0], hb) + ones_row
        s_row = jnp.dot(A1s_ref[pl.ds(h, 1)][0], whxT,
                        preferred_element_type=jnp.float32)
        d_row = jnp.dot(A1d_ref[pl.ds(h, 1)][0], whxT,
                        preferred_element_type=jnp.float32)
        s_col = jnp.dot(hb, W1as_ref[pl.ds(h, 1)][0],
                        preferred_element_type=jnp.float32)
        store_factors(whxT, s_row, d_row, s_col)

    @pl.when(jnp.logical_and(pid >= NJ, pid < 2 * NJ))
    def _phase_attn1():
        attn(r, True)

    @pl.when(jnp.logical_and(pid >= 2 * NJ, pid < 3 * NJ))
    def _phase_prep2():
        parts = [h1_scr[pl.ds(jj, 1), pl.ds(b, 1)][0, 0] for jj in range(NJ)]
        hbT = jnp.concatenate(parts, axis=1)
        whxT = jnp.dot(W2eT_ref[pl.ds(h, 1)][0], hbT,
                       preferred_element_type=jnp.float32) + ones_row
        s_row = jnp.dot(A2s_ref[pl.ds(h, 1)][0], whxT,
                        preferred_element_type=jnp.float32)
        d_row = jnp.dot(A2d_ref[pl.ds(h, 1)][0], whxT,
                        preferred_element_type=jnp.float32)
        s_col = jnp.reshape(s_row, (NUM_NODES, 1))
        store_factors(whxT, s_row, d_row, s_col)

    @pl.when(pid >= 3 * NJ)
    def _phase_attn2():
        attn(r, False)


def kernel(x, embedding, proj_W, proj_b, W1, a1_src, a1_dst, b1,
           W2, a2_src, a2_dst, b2):
    N = NUM_NODES
    W1eT, W2eT = _wext_t(W1), _wext_t(W2)
    A1s = jnp.pad(a1_src, ((0, 0), (0, 1)))[:, None, :]
    A2s = jnp.pad(a2_src, ((0, 0), (0, 1)))[:, None, :]
    A1d = jnp.pad(a1_dst, ((0, 0), (0, 1)))[:, None, :]
    A2d = jnp.pad(a2_dst, ((0, 0), (0, 1)))[:, None, :]
    W1as = jnp.einsum('khf,hf->hk', W1.reshape(IN_DIM, HEADS, F), a1_src)
    W1as1 = W1as[:, :, None]

    full = lambda *shape: pl.BlockSpec(shape, lambda p: (0,) * len(shape))
    out_t = pl.pallas_call(
        _fused_body,
        grid=(4 * NJ,),
        in_specs=[
            full(N, IN_DIM),
            full(BATCH, N, IN_DIM),
            full(IN_DIM, IN_DIM),
            full(IN_DIM),
            full(HEADS, FE, IN_DIM),
            full(HEADS, 1, FE),
            full(HEADS, 1, FE),
            full(HEADS, IN_DIM, 1),
            full(HEADS, FE, HIDDEN_DIM),
            full(HEADS, 1, FE),
            full(HEADS, 1, FE),
            full(HIDDEN_DIM, 1),
            full(HIDDEN_DIM, 1),
        ],
        out_specs=pl.BlockSpec(
            (BATCH, HIDDEN_DIM, JB),
            lambda p: (0, 0, jnp.maximum(p - 3 * NJ, 0))),
        out_shape=jax.ShapeDtypeStruct((BATCH, HIDDEN_DIM, N), jnp.float32),
        scratch_shapes=[
            pltpu.VMEM((NJ, N, JB), jnp.bfloat16),
            pltpu.VMEM((BATCH, HEADS, FE, N), jnp.bfloat16),
            pltpu.VMEM((BATCH, HEADS, FE, N), jnp.bfloat16),
            pltpu.VMEM((BATCH, HEADS, N, 1), jnp.float32),
            pltpu.VMEM((BATCH, HEADS, NJ, JB), jnp.float32),
            pltpu.VMEM((BATCH, HEADS, NJ, JB), jnp.float32),
            pltpu.VMEM((BATCH, HEADS, NJ, JB), jnp.float32),
            pltpu.VMEM((NJ, BATCH, HIDDEN_DIM, JB), jnp.float32),
            pltpu.VMEM((N, IN_DIM), jnp.float32),
        ],
    )(embedding, x, proj_W, proj_b,
      W1eT, A1s, A1d, W1as1,
      W2eT, A2s, A2d,
      b1[:, None], b2[:, None])
    return jnp.swapaxes(out_t, 1, 2)
